# baseline (speedup 1.0000x reference)
import functools

import jax
import jax.numpy as jnp
from jax import lax
from jax.experimental import pallas as pl
from jax.experimental.pallas import tpu as pltpu
from jax.experimental.pallas import tpu_sc as plsc

B = 16384
D = 16
NC = 2
NS = 16
NW = NC * NS
BPW = B // NW

_mesh = plsc.VectorSubcoreMesh(core_axis_name="c", subcore_axis_name="s")


@functools.partial(
    pl.kernel,
    mesh=_mesh,
    compiler_params=pltpu.CompilerParams(
        needs_layout_passes=False, use_tc_tiling_on_sc=False),
    out_type=jax.ShapeDtypeStruct((B,), jnp.float32),
    scratch_types=[
        pltpu.VMEM((BPW,), jnp.int32),
        pltpu.VMEM((BPW,), jnp.int32),
        pltpu.VMEM((BPW, D), jnp.float32),
        pltpu.VMEM((BPW, D), jnp.float32),
        pltpu.VMEM((BPW,), jnp.float32),
        pltpu.SemaphoreType.DMA,
        pltpu.SemaphoreType.DMA,
    ],
)
def _mf_sc(xu_hbm, xi_hbm, user_hbm, item_hbm, out_hbm,
           idxu_v, idxi_v, urows_v, irows_v, out_v, sem_u, sem_i):
    wid = lax.axis_index("s") * NC + lax.axis_index("c")
    base = wid * BPW
    pltpu.sync_copy(xu_hbm.at[pl.ds(base, BPW)], idxu_v)
    pltpu.sync_copy(xi_hbm.at[pl.ds(base, BPW)], idxi_v)
    cu = pltpu.async_copy(user_hbm.at[idxu_v], urows_v, sem_u)
    ci = pltpu.async_copy(item_hbm.at[idxi_v], irows_v, sem_i)
    cu.wait()
    ci.wait()

    lanes = lax.iota(jnp.int32, D)

    def body(blk, carry):
        row0 = blk * D
        acc = jnp.zeros((D,), jnp.float32)
        for j in range(D):
            w = urows_v[row0 + j] * irows_v[row0 + j]
            s = jnp.sum(w)
            acc = jnp.where(lanes == j, s, acc)
        out_v[pl.ds(row0, D)] = acc
        return carry

    lax.fori_loop(0, BPW // D, body, 0)
    pltpu.sync_copy(out_v, out_hbm.at[pl.ds(base, BPW)])


def kernel(X, user_emb, item_emb):
    xu = X[:, 0]
    xi = X[:, 1]
    out = _mf_sc(xu, xi, user_emb, item_emb)
    return out.reshape(B, 1)
